# baseline (speedup 1.0000x reference)
import numpy as np
import jax
import jax.numpy as jnp
from jax.experimental import pallas as pl

_ALPHA = 4


def _slow_idx(t: int) -> list:
    n = t // _ALPHA
    return [int(v) for v in np.linspace(0.0, t - 1, n).astype(np.int32)]


def kernel(frames):
    c, t, h, w = frames.shape
    idx = _slow_idx(t)
    n = len(idx)
    hw = h * w
    g_sz = t // n
    offs = [idx[g] - g_sz * g for g in range(n)]
    assert all(0 <= o < g_sz for o in offs)

    rows = frames.reshape(c * t, 1, hw)

    def body(in_ref, slow_ref, fast_ref):
        j = pl.program_id(0)
        g = jax.lax.rem(j, n)
        fast_ref[...] = in_ref[...]
        off = jnp.int32(offs[0])
        for k in range(1, n):
            off = jnp.where(g == k, jnp.int32(offs[k]), off)
        slow_ref[...] = in_ref[pl.ds(off, 1)]

    slow, fast = pl.pallas_call(
        body,
        grid=(c * n,),
        in_specs=[pl.BlockSpec((g_sz, 1, hw), lambda j: (j, 0, 0))],
        out_specs=[
            pl.BlockSpec((1, 1, hw), lambda j: (j, 0, 0)),
            pl.BlockSpec((g_sz, 1, hw), lambda j: (j, 0, 0)),
        ],
        out_shape=[
            jax.ShapeDtypeStruct((c * n, 1, hw), frames.dtype),
            jax.ShapeDtypeStruct((c * t, 1, hw), frames.dtype),
        ],
    )(rows)
    return (slow.reshape(c, n, h, w), fast.reshape(c, t, h, w))


# speedup vs baseline: 15.2477x; 15.2477x over previous
import numpy as np
import jax
import jax.numpy as jnp
from jax.experimental import pallas as pl

_ALPHA = 4


def _slow_idx(t: int) -> list:
    n = t // _ALPHA
    return [int(v) for v in np.linspace(0.0, t - 1, n).astype(np.int32)]


def kernel(frames):
    c, t, h, w = frames.shape
    idx = _slow_idx(t)
    n = len(idx)
    g_sz = t // n
    offs = [idx[g] - g_sz * g for g in range(n)]
    assert all(0 <= o < g_sz for o in offs)

    def body(in_ref, slow_ref, fast_ref):
        j = pl.program_id(0)
        g = jax.lax.rem(j, n)
        fast_ref[...] = in_ref[...]
        off = jnp.int32(offs[0])
        for k in range(1, n):
            off = jnp.where(g == k, jnp.int32(offs[k]), off)
        slow_ref[...] = in_ref[:, pl.ds(off, 1)]

    slow, fast = pl.pallas_call(
        body,
        grid=(c * n,),
        in_specs=[
            pl.BlockSpec((1, g_sz, h, w), lambda j: (j // n, j % n, 0, 0))
        ],
        out_specs=[
            pl.BlockSpec((1, 1, h, w), lambda j: (j // n, j % n, 0, 0)),
            pl.BlockSpec((1, g_sz, h, w), lambda j: (j // n, j % n, 0, 0)),
        ],
        out_shape=[
            jax.ShapeDtypeStruct((c, n, h, w), frames.dtype),
            jax.ShapeDtypeStruct((c, t, h, w), frames.dtype),
        ],
    )(frames)
    return (slow, fast)
